# baseline (speedup 1.0000x reference)
import functools

import jax
import jax.numpy as jnp
from jax import lax
from jax.experimental import pallas as pl
from jax.experimental.pallas import tpu as pltpu
from jax.experimental.pallas import tpu_sc as plsc


@functools.lru_cache(maxsize=None)
def _make_sc_pair_gather(V2, D2, B):
    info = plsc.get_sparse_core_info()
    NC, NS = info.num_cores, info.num_subcores
    NW = NC * NS
    assert B % NW == 0 and (B // NW) % 8 == 0
    b_per_w = B // NW
    mesh = plsc.VectorSubcoreMesh(core_axis_name="c", subcore_axis_name="s")

    @functools.partial(
        pl.kernel,
        mesh=mesh,
        out_type=jax.ShapeDtypeStruct((B, D2), jnp.float32),
        scratch_types=[
            pltpu.VMEM((b_per_w,), jnp.int32),
            pltpu.VMEM((b_per_w, D2), jnp.float32),
            pltpu.SemaphoreType.DMA,
        ],
    )
    def gather_k(idx_hbm, table_hbm, out_hbm, idx_v, rows_v, sem):
        wid = lax.axis_index("s") * NC + lax.axis_index("c")
        base = wid * b_per_w
        pltpu.sync_copy(idx_hbm.at[pl.ds(base, b_per_w)], idx_v)
        for k in range(b_per_w // 16):
            sl = pl.ds(k * 16, 16)
            idx_v[sl] = lax.shift_right_logical(idx_v[sl], 1)
        pltpu.async_copy(table_hbm.at[idx_v], rows_v, sem).wait()
        pltpu.sync_copy(rows_v, out_hbm.at[pl.ds(base, b_per_w)])

    return gather_k


@functools.lru_cache(maxsize=None)
def _make_tc_matmul(B, D, V, MS, TN):
    NSLAB = B // MS
    NT = pl.cdiv(V, TN)
    assert NSLAB >= 3 and MS % 8 == 0

    def body(x2_hbm, ids_hbm, hbf_hbm, o_hbm,
             hbf_v, x2_v, ids_v, xsel_v, o_s,
             sem_h, sem_x, sem_i, sem_o):
        m = pl.program_id(0)
        buf = lax.rem(m, 2)

        def o_cp(slot, base):
            return pltpu.make_async_copy(
                o_s.at[slot], o_hbm.at[pl.ds(base, MS)], sem_o.at[slot])

        @pl.when(m == 0)
        def _prologue():
            ch = pltpu.make_async_copy(hbf_hbm, hbf_v, sem_h)
            cx = pltpu.make_async_copy(x2_hbm, x2_v, sem_x)
            ci = pltpu.make_async_copy(ids_hbm, ids_v, sem_i)
            ch.start()
            cx.start()
            ci.start()
            cx.wait()
            ci.wait()
            for i in range(NSLAB):
                rows = pl.ds(i * MS, MS)
                odd = (ids_v[rows] & 1) == 1
                xsel_v[i] = jnp.where(
                    odd, x2_v[rows, D:2 * D], x2_v[rows, :D]
                ).astype(jnp.bfloat16)
            ch.wait()

        @pl.when(m >= 2)
        def _drain_prev():
            o_cp(buf, m * MS).wait()

        xs = xsel_v[m]
        for jt in range(NT - 1):
            o_s[buf, :, jt * TN:(jt + 1) * TN] = lax.dot_general(
                xs, hbf_v[jt * TN:(jt + 1) * TN, :],
                dimension_numbers=(((1,), (1,)), ((), ())),
                preferred_element_type=jnp.float32)
        o_s[buf, :, (NT - 1) * TN:V] = lax.dot_general(
            xs, hbf_v[(NT - 1) * TN:V, :],
            dimension_numbers=(((1,), (1,)), ((), ())),
            preferred_element_type=jnp.float32)

        o_cp(buf, m * MS).start()

        @pl.when(m == NSLAB - 1)
        def _final_drain():
            o_cp(buf, m * MS).wait()
            o_cp(1 - buf, m * MS).wait()

    return pl.pallas_call(
        body,
        grid=(NSLAB,),
        in_specs=[
            pl.BlockSpec(memory_space=pltpu.MemorySpace.HBM),
            pl.BlockSpec(memory_space=pltpu.MemorySpace.HBM),
            pl.BlockSpec(memory_space=pltpu.MemorySpace.HBM),
        ],
        out_specs=pl.BlockSpec(memory_space=pltpu.MemorySpace.HBM),
        out_shape=jax.ShapeDtypeStruct((B, V), jnp.float32),
        scratch_shapes=[
            pltpu.VMEM((V, D), jnp.bfloat16),
            pltpu.VMEM((B, 2 * D), jnp.float32),
            pltpu.VMEM((B, 1), jnp.int32),
            pltpu.VMEM((B // MS, MS, D), jnp.bfloat16),
            pltpu.VMEM((2, MS, V), jnp.float32),
            pltpu.SemaphoreType.DMA,
            pltpu.SemaphoreType.DMA,
            pltpu.SemaphoreType.DMA,
            pltpu.SemaphoreType.DMA((2,)),
        ],
        compiler_params=pltpu.CompilerParams(
            dimension_semantics=("arbitrary",),
        ),
    )


def kernel(input_ids, embed_weight, head_weight):
    B = input_ids.shape[0]
    V, D = embed_weight.shape
    ids = input_ids.astype(jnp.int32)
    table2 = embed_weight.reshape(V // 2, 2 * D)
    x2 = _make_sc_pair_gather(V // 2, 2 * D, B)(ids, table2)
    head_bf = head_weight.astype(jnp.bfloat16)
    return _make_tc_matmul(B, D, V, 32, 1024)(
        x2, ids.reshape(B, 1), head_bf)


# speedup vs baseline: 1.0026x; 1.0026x over previous
import functools

import jax
import jax.numpy as jnp
from jax import lax
from jax.experimental import pallas as pl
from jax.experimental.pallas import tpu as pltpu
from jax.experimental.pallas import tpu_sc as plsc


@functools.lru_cache(maxsize=None)
def _make_sc_pair_gather(V2, D2, B):
    info = plsc.get_sparse_core_info()
    NC, NS = info.num_cores, info.num_subcores
    NW = NC * NS
    assert B % NW == 0 and (B // NW) % 8 == 0
    b_per_w = B // NW
    mesh = plsc.VectorSubcoreMesh(core_axis_name="c", subcore_axis_name="s")

    @functools.partial(
        pl.kernel,
        mesh=mesh,
        out_type=jax.ShapeDtypeStruct((B, D2), jnp.float32),
        scratch_types=[
            pltpu.VMEM((b_per_w,), jnp.int32),
            pltpu.VMEM((b_per_w, D2), jnp.float32),
            pltpu.SemaphoreType.DMA,
        ],
    )
    def gather_k(idx_hbm, table_hbm, out_hbm, idx_v, rows_v, sem):
        wid = lax.axis_index("s") * NC + lax.axis_index("c")
        base = wid * b_per_w
        pltpu.sync_copy(idx_hbm.at[pl.ds(base, b_per_w)], idx_v)
        for k in range(b_per_w // 16):
            sl = pl.ds(k * 16, 16)
            idx_v[sl] = lax.shift_right_logical(idx_v[sl], 1)
        pltpu.async_copy(table_hbm.at[idx_v], rows_v, sem).wait()
        pltpu.sync_copy(rows_v, out_hbm.at[pl.ds(base, b_per_w)])

    return gather_k


@functools.lru_cache(maxsize=None)
def _make_tc_matmul(B, D, V, MS, TN):
    NSLAB = B // MS
    NT = pl.cdiv(V, TN)
    assert NSLAB >= 3 and MS % 8 == 0

    def body(x2_hbm, ids_hbm, hbf_hbm, o_hbm,
             hbf_v, x2_v, ids_v, xsel_v, o_s,
             sem_h, sem_x, sem_i, sem_o):
        m = pl.program_id(0)
        buf = lax.rem(m, 2)

        def o_cp(slot, base):
            return pltpu.make_async_copy(
                o_s.at[slot], o_hbm.at[pl.ds(base, MS)], sem_o.at[slot])

        @pl.when(m == 0)
        def _prologue():
            ch = pltpu.make_async_copy(hbf_hbm, hbf_v, sem_h)
            cx = pltpu.make_async_copy(x2_hbm, x2_v, sem_x)
            ci = pltpu.make_async_copy(ids_hbm, ids_v, sem_i)
            ch.start()
            cx.start()
            ci.start()
            cx.wait()
            ci.wait()
            for i in range(NSLAB):
                rows = pl.ds(i * MS, MS)
                odd = (ids_v[rows] & 1) == 1
                xsel_v[i] = jnp.where(
                    odd, x2_v[rows, D:2 * D], x2_v[rows, :D]
                ).astype(jnp.bfloat16)
            ch.wait()

        @pl.when(m >= 2)
        def _drain_prev():
            o_cp(buf, m * MS).wait()

        xs = xsel_v[m]
        for jt in range(NT - 1):
            o_s[buf, :, jt * TN:(jt + 1) * TN] = lax.dot_general(
                xs, hbf_v[jt * TN:(jt + 1) * TN, :],
                dimension_numbers=(((1,), (1,)), ((), ())),
                preferred_element_type=jnp.float32)
        o_s[buf, :, (NT - 1) * TN:V] = lax.dot_general(
            xs, hbf_v[(NT - 1) * TN:V, :],
            dimension_numbers=(((1,), (1,)), ((), ())),
            preferred_element_type=jnp.float32)

        o_cp(buf, m * MS).start()

        @pl.when(m == NSLAB - 1)
        def _final_drain():
            o_cp(buf, m * MS).wait()
            o_cp(1 - buf, m * MS).wait()

    return pl.pallas_call(
        body,
        grid=(NSLAB,),
        in_specs=[
            pl.BlockSpec(memory_space=pltpu.MemorySpace.HBM),
            pl.BlockSpec(memory_space=pltpu.MemorySpace.HBM),
            pl.BlockSpec(memory_space=pltpu.MemorySpace.HBM),
        ],
        out_specs=pl.BlockSpec(memory_space=pltpu.MemorySpace.HBM),
        out_shape=jax.ShapeDtypeStruct((B, V), jnp.float32),
        scratch_shapes=[
            pltpu.VMEM((V, D), jnp.bfloat16),
            pltpu.VMEM((B, 2 * D), jnp.float32),
            pltpu.VMEM((B, 1), jnp.int32),
            pltpu.VMEM((B // MS, MS, D), jnp.bfloat16),
            pltpu.VMEM((2, MS, V), jnp.float32),
            pltpu.SemaphoreType.DMA,
            pltpu.SemaphoreType.DMA,
            pltpu.SemaphoreType.DMA,
            pltpu.SemaphoreType.DMA((2,)),
        ],
        compiler_params=pltpu.CompilerParams(
            dimension_semantics=("arbitrary",),
        ),
    )


def kernel(input_ids, embed_weight, head_weight):
    B = input_ids.shape[0]
    V, D = embed_weight.shape
    ids = input_ids.astype(jnp.int32)
    table2 = embed_weight.reshape(V // 2, 2 * D)
    x2 = _make_sc_pair_gather(V // 2, 2 * D, B)(ids, table2)
    head_bf = head_weight.astype(jnp.bfloat16)
    return _make_tc_matmul(B, D, V, 32, 8192)(
        x2, ids.reshape(B, 1), head_bf)


# speedup vs baseline: 1.0287x; 1.0261x over previous
import functools

import jax
import jax.numpy as jnp
from jax import lax
from jax.experimental import pallas as pl
from jax.experimental.pallas import tpu as pltpu
from jax.experimental.pallas import tpu_sc as plsc


@functools.lru_cache(maxsize=None)
def _make_sc_pair_gather(V2, D2, B):
    info = plsc.get_sparse_core_info()
    NC, NS = info.num_cores, info.num_subcores
    NW = NC * NS
    assert B % NW == 0 and (B // NW) % 8 == 0
    b_per_w = B // NW
    mesh = plsc.VectorSubcoreMesh(core_axis_name="c", subcore_axis_name="s")

    @functools.partial(
        pl.kernel,
        mesh=mesh,
        out_type=jax.ShapeDtypeStruct((B, D2), jnp.float32),
        scratch_types=[
            pltpu.VMEM((b_per_w,), jnp.int32),
            pltpu.VMEM((b_per_w, D2), jnp.float32),
            pltpu.SemaphoreType.DMA,
        ],
    )
    def gather_k(idx_hbm, table_hbm, out_hbm, idx_v, rows_v, sem):
        wid = lax.axis_index("s") * NC + lax.axis_index("c")
        base = wid * b_per_w
        pltpu.sync_copy(idx_hbm.at[pl.ds(base, b_per_w)], idx_v)
        for k in range(b_per_w // 16):
            sl = pl.ds(k * 16, 16)
            idx_v[sl] = lax.shift_right_logical(idx_v[sl], 1)
        pltpu.async_copy(table_hbm.at[idx_v], rows_v, sem).wait()
        pltpu.sync_copy(rows_v, out_hbm.at[pl.ds(base, b_per_w)])

    return gather_k


@functools.lru_cache(maxsize=None)
def _make_tc_matmul(B, D, V, MS, TN):
    NSLAB = B // MS
    NT = pl.cdiv(V, TN)
    assert NSLAB >= 3 and MS % 8 == 0

    def body(x2_hbm, ids_hbm, hbf_hbm, o_hbm,
             hbf_v, x2_v, ids_v, xsel_v, o_s0, o_s1,
             sem_h, sem_x, sem_i, sem_o):
        m = pl.program_id(0)
        buf = lax.rem(m, 2)

        def o_cp(slot, base):
            o_ref = o_s0 if slot == 0 else o_s1
            return pltpu.make_async_copy(
                o_ref, o_hbm.at[pl.ds(base, MS)], sem_o.at[slot])

        @pl.when(m == 0)
        def _prologue():
            ch = pltpu.make_async_copy(hbf_hbm, hbf_v, sem_h)
            cx = pltpu.make_async_copy(x2_hbm, x2_v, sem_x)
            ci = pltpu.make_async_copy(ids_hbm, ids_v, sem_i)
            ch.start()
            cx.start()
            ci.start()
            cx.wait()
            ci.wait()
            for i in range(NSLAB):
                rows = pl.ds(i * MS, MS)
                odd = (ids_v[rows] & 1) == 1
                xsel_v[i] = jnp.where(
                    odd, x2_v[rows, D:2 * D], x2_v[rows, :D]
                ).astype(jnp.bfloat16)
            ch.wait()

        def compute_into(o_ref):
            xs = xsel_v[m]
            for jt in range(NT - 1):
                o_ref[:, jt * TN:(jt + 1) * TN] = lax.dot_general(
                    xs, hbf_v[jt * TN:(jt + 1) * TN, :],
                    dimension_numbers=(((1,), (1,)), ((), ())),
                    preferred_element_type=jnp.float32)
            o_ref[:, (NT - 1) * TN:V] = lax.dot_general(
                xs, hbf_v[(NT - 1) * TN:V, :],
                dimension_numbers=(((1,), (1,)), ((), ())),
                preferred_element_type=jnp.float32)

        @pl.when(buf == 0)
        def _even_slab():
            @pl.when(m >= 2)
            def _():
                o_cp(0, m * MS).wait()
            compute_into(o_s0)
            o_cp(0, m * MS).start()

        @pl.when(buf == 1)
        def _odd_slab():
            @pl.when(m >= 2)
            def _():
                o_cp(1, m * MS).wait()
            compute_into(o_s1)
            o_cp(1, m * MS).start()

        @pl.when(m == NSLAB - 1)
        def _final_drain():
            o_cp(0, m * MS).wait()
            o_cp(1, m * MS).wait()

    return pl.pallas_call(
        body,
        grid=(NSLAB,),
        in_specs=[
            pl.BlockSpec(memory_space=pltpu.MemorySpace.HBM),
            pl.BlockSpec(memory_space=pltpu.MemorySpace.HBM),
            pl.BlockSpec(memory_space=pltpu.MemorySpace.HBM),
        ],
        out_specs=pl.BlockSpec(memory_space=pltpu.MemorySpace.HBM),
        out_shape=jax.ShapeDtypeStruct((B, V), jnp.float32),
        scratch_shapes=[
            pltpu.VMEM((V, D), jnp.bfloat16),
            pltpu.VMEM((B, 2 * D), jnp.float32),
            pltpu.VMEM((B, 1), jnp.int32),
            pltpu.VMEM((B // MS, MS, D), jnp.bfloat16),
            pltpu.VMEM((MS, V), jnp.float32),
            pltpu.VMEM((MS, V), jnp.float32),
            pltpu.SemaphoreType.DMA,
            pltpu.SemaphoreType.DMA,
            pltpu.SemaphoreType.DMA,
            pltpu.SemaphoreType.DMA((2,)),
        ],
        compiler_params=pltpu.CompilerParams(
            dimension_semantics=("arbitrary",),
        ),
    )


def kernel(input_ids, embed_weight, head_weight):
    B = input_ids.shape[0]
    V, D = embed_weight.shape
    ids = input_ids.astype(jnp.int32)
    table2 = embed_weight.reshape(V // 2, 2 * D)
    x2 = _make_sc_pair_gather(V // 2, 2 * D, B)(ids, table2)
    head_bf = head_weight.astype(jnp.bfloat16)
    return _make_tc_matmul(B, D, V, 32, 8192)(
        x2, ids.reshape(B, 1), head_bf)


# speedup vs baseline: 1.2438x; 1.2091x over previous
import functools

import jax
import jax.numpy as jnp
from jax import lax
from jax.experimental import pallas as pl
from jax.experimental.pallas import tpu as pltpu
from jax.experimental.pallas import tpu_sc as plsc


@functools.lru_cache(maxsize=None)
def _make_sc_pair_gather(V2, D2, B):
    info = plsc.get_sparse_core_info()
    NC, NS = info.num_cores, info.num_subcores
    NW = NC * NS
    assert B % NW == 0 and (B // NW) % 8 == 0
    b_per_w = B // NW
    mesh = plsc.VectorSubcoreMesh(core_axis_name="c", subcore_axis_name="s")

    @functools.partial(
        pl.kernel,
        mesh=mesh,
        out_type=jax.ShapeDtypeStruct((B, D2), jnp.float32),
        scratch_types=[
            pltpu.VMEM((b_per_w,), jnp.int32),
            pltpu.VMEM((b_per_w, D2), jnp.float32),
            pltpu.SemaphoreType.DMA,
        ],
    )
    def gather_k(idx_hbm, table_hbm, out_hbm, idx_v, rows_v, sem):
        wid = lax.axis_index("s") * NC + lax.axis_index("c")
        base = wid * b_per_w
        pltpu.sync_copy(idx_hbm.at[pl.ds(base, b_per_w)], idx_v)
        for k in range(b_per_w // 16):
            sl = pl.ds(k * 16, 16)
            idx_v[sl] = lax.shift_right_logical(idx_v[sl], 1)
        pltpu.async_copy(table_hbm.at[idx_v], rows_v, sem).wait()
        pltpu.sync_copy(rows_v, out_hbm.at[pl.ds(base, b_per_w)])

    return gather_k


@functools.lru_cache(maxsize=None)
def _make_tc_matmul(B, D, V, TN, KS):
    NB = pl.cdiv(V, TN)
    RS = B // KS
    TAIL = V - (NB - 1) * TN
    assert NB >= 3 and TAIL % 8 == 0 and (NB - 1) * TN % 128 == 0

    def body(x2_hbm, ids_hbm, h_hbm, o_hbm,
             x2_v, ids_v, xsel_v, h_v, o_v, o_tail_v,
             sem_x, sem_i, sem_h, sem_o):
        j = pl.program_id(0)
        nb = pl.num_programs(0)
        buf = lax.rem(j, 2)

        def h_copy(slot, base, width):
            return pltpu.make_async_copy(
                h_hbm.at[pl.ds(base, width)],
                h_v.at[slot, pl.ds(0, width)],
                sem_h.at[slot])

        def o_copy(slot, s, base, width):
            return pltpu.make_async_copy(
                o_v.at[slot, pl.ds(s * RS, RS), pl.ds(0, width)],
                o_hbm.at[pl.ds(s * RS, RS), pl.ds(base, width)],
                sem_o.at[slot, s])

        def o_tail_copy(s):
            return pltpu.make_async_copy(
                o_tail_v.at[pl.ds(s * RS, RS)],
                o_hbm.at[pl.ds(s * RS, RS), pl.ds((NB - 1) * TN, TAIL)],
                sem_o.at[lax.rem(NB - 1, 2), s])

        @pl.when(j == 0)
        def _prologue():
            cx = pltpu.make_async_copy(x2_hbm, x2_v, sem_x)
            ci = pltpu.make_async_copy(ids_hbm, ids_v, sem_i)
            cx.start()
            ci.start()
            h_copy(0, 0, TN).start()
            cx.wait()
            ci.wait()
            odd = (ids_v[...] & 1) == 1
            xsel_v[...] = jnp.where(
                odd, x2_v[:, D:2 * D], x2_v[:, :D]).astype(jnp.bfloat16)

        @pl.when(j + 1 < nb - 1)
        def _prefetch_full():
            h_copy(lax.rem(j + 1, 2), (j + 1) * TN, TN).start()

        @pl.when(j + 1 == nb - 1)
        def _prefetch_tail():
            h_copy(lax.rem(j + 1, 2), (nb - 1) * TN, TAIL).start()

        @pl.when(j < nb - 1)
        def _wait_full():
            h_copy(buf, j * TN, TN).wait()

        @pl.when(j == nb - 1)
        def _wait_tail():
            h_copy(buf, (nb - 1) * TN, TAIL).wait()

        @pl.when(j >= 2)
        def _drain_prev():
            for s in range(KS):
                o_copy(buf, s, 0, TN).wait()

        @pl.when(j < nb - 1)
        def _compute_and_issue_full():
            o_v[buf] = lax.dot_general(
                xsel_v[...], h_v[buf],
                dimension_numbers=(((1,), (1,)), ((), ())),
                preferred_element_type=jnp.float32)
            for s in range(KS):
                o_copy(buf, s, j * TN, TN).start()

        @pl.when(j == nb - 1)
        def _compute_tail_and_drain_all():
            o_tail_v[...] = lax.dot_general(
                xsel_v[...], h_v[buf, pl.ds(0, TAIL)],
                dimension_numbers=(((1,), (1,)), ((), ())),
                preferred_element_type=jnp.float32)
            for s in range(KS):
                o_tail_copy(s).start()
            for s in range(KS):
                o_tail_copy(s).wait()
            for s in range(KS):
                o_copy(1 - buf, s, 0, TN).wait()

    return pl.pallas_call(
        body,
        grid=(NB,),
        in_specs=[
            pl.BlockSpec(memory_space=pltpu.MemorySpace.HBM),
            pl.BlockSpec(memory_space=pltpu.MemorySpace.HBM),
            pl.BlockSpec(memory_space=pltpu.MemorySpace.HBM),
        ],
        out_specs=pl.BlockSpec(memory_space=pltpu.MemorySpace.HBM),
        out_shape=jax.ShapeDtypeStruct((B, V), jnp.float32),
        scratch_shapes=[
            pltpu.VMEM((B, 2 * D), jnp.float32),
            pltpu.VMEM((B, 1), jnp.int32),
            pltpu.VMEM((B, D), jnp.bfloat16),
            pltpu.VMEM((2, TN, D), jnp.bfloat16),
            pltpu.VMEM((2, B, TN), jnp.float32),
            pltpu.VMEM((B, TAIL), jnp.float32),
            pltpu.SemaphoreType.DMA,
            pltpu.SemaphoreType.DMA,
            pltpu.SemaphoreType.DMA((2,)),
            pltpu.SemaphoreType.DMA((2, KS)),
        ],
        compiler_params=pltpu.CompilerParams(
            dimension_semantics=("arbitrary",),
        ),
    )


def kernel(input_ids, embed_weight, head_weight):
    B = input_ids.shape[0]
    V, D = embed_weight.shape
    ids = input_ids.astype(jnp.int32)
    table2 = embed_weight.reshape(V // 2, 2 * D)
    x2 = _make_sc_pair_gather(V // 2, 2 * D, B)(ids, table2)
    head_bf = head_weight.astype(jnp.bfloat16)
    return _make_tc_matmul(B, D, V, 1024, 8)(
        x2, ids.reshape(B, 1), head_bf)
